# Initial kernel scaffold; baseline (speedup 1.0000x reference)
import jax
import jax.numpy as jnp
from jax.experimental import pallas as pl


def kernel(x, Wg, W1, b1, W2, b2):
    raise NotImplementedError("write your pallas kernel here")


# speedup vs baseline: 1.0296x; 1.0296x over previous
import functools

import jax
import jax.numpy as jnp
from jax import lax
from jax.experimental import pallas as pl
from jax.experimental.pallas import tpu as pltpu
from jax.experimental.pallas import tpu_sc as plsc

HID = 1024
FF = 4096
E = 8
T = 4096
C = T // E
CHUNK = 512
NCHUNK = T // CHUNK
NTILES = 32
TPT = T // NTILES
DISP_ROWS = T + NTILES
FBLK = 512
NF = FF // FBLK


def _routing_body(x_ref, wg_ref, dest_ref, src_ref, gate_ref, cnt_ref):
    pid = pl.program_id(0)

    @pl.when(pid == 0)
    def _():
        cnt_ref[...] = jnp.zeros_like(cnt_ref)

    xb = x_ref[...].astype(jnp.bfloat16)
    wg = wg_ref[...].astype(jnp.bfloat16)
    logits = jnp.dot(xb, wg, preferred_element_type=jnp.float32)
    m = jnp.max(logits, axis=1, keepdims=True)
    gate = 1.0 / jnp.sum(jnp.exp(logits - m), axis=1, keepdims=True)
    iota_e = lax.broadcasted_iota(jnp.int32, (CHUNK, E), 1)
    eidx = jnp.min(jnp.where(logits == m, iota_e, E), axis=1, keepdims=True)
    maskf = (iota_e == eidx).astype(jnp.float32)

    r = lax.broadcasted_iota(jnp.int32, (CHUNK, CHUNK), 0)
    c = lax.broadcasted_iota(jnp.int32, (CHUNK, CHUNK), 1)
    l16 = (c < r).astype(jnp.bfloat16)
    locin = jnp.dot(l16, maskf.astype(jnp.bfloat16),
                    preferred_element_type=jnp.float32)
    base = cnt_ref[0:1, 0:E]
    locf = jnp.sum((locin + base) * maskf, axis=1, keepdims=True)
    cnt_ref[0:1, 0:E] = base + jnp.sum(maskf, axis=0, keepdims=True)

    keep = locf < float(C)
    loc = locf.astype(jnp.int32)
    locc = jnp.minimum(loc, C - 1)
    tok = pid * CHUNK + lax.broadcasted_iota(jnp.int32, (CHUNK, 1), 0)
    trash = T + tok // TPT
    dest_ref[...] = jnp.where(keep, eidx * C + loc, trash)
    src_ref[...] = eidx * C + locc
    gate_ref[...] = jnp.broadcast_to(gate * keep.astype(jnp.float32), (CHUNK, 16))


def _routing(xt, Wg):
    out_shape = (
        jax.ShapeDtypeStruct((T, 1), jnp.int32),
        jax.ShapeDtypeStruct((T, 1), jnp.int32),
        jax.ShapeDtypeStruct((T, 16), jnp.float32),
    )
    dest, src, gate_rep = pl.pallas_call(
        _routing_body,
        grid=(NCHUNK,),
        in_specs=[
            pl.BlockSpec((CHUNK, HID), lambda i: (i, 0)),
            pl.BlockSpec((HID, E), lambda i: (0, 0)),
        ],
        out_specs=(
            pl.BlockSpec((CHUNK, 1), lambda i: (i, 0)),
            pl.BlockSpec((CHUNK, 1), lambda i: (i, 0)),
            pl.BlockSpec((CHUNK, 16), lambda i: (i, 0)),
        ),
        out_shape=out_shape,
        scratch_shapes=[pltpu.VMEM((8, 128), jnp.float32)],
    )(xt, Wg)
    return dest.reshape(T), src.reshape(T), gate_rep


_DCHUNK = 64


@functools.cache
def _sc_mesh():
    return plsc.VectorSubcoreMesh(core_axis_name="c", subcore_axis_name="s")


@functools.cache
def _dispatch_sc():
    @functools.partial(
        pl.kernel,
        mesh=_sc_mesh(),
        out_type=jax.ShapeDtypeStruct((DISP_ROWS, HID), jnp.float32),
        scratch_types=[
            pltpu.VMEM((_DCHUNK,), jnp.int32),
            pltpu.VMEM((_DCHUNK, HID), jnp.float32),
            pltpu.SemaphoreType.DMA,
        ],
    )
    def dispatch(xt_hbm, dest_hbm, disp_hbm, idx_v, rows_v, sem):
        wid = lax.axis_index("s") * 2 + lax.axis_index("c")
        for k in range(TPT // _DCHUNK):
            base = wid * TPT + k * _DCHUNK
            pltpu.sync_copy(xt_hbm.at[pl.ds(base, _DCHUNK)], rows_v)
            pltpu.sync_copy(dest_hbm.at[pl.ds(base, _DCHUNK)], idx_v)
            pltpu.async_copy(rows_v, disp_hbm.at[idx_v], sem).wait()

    return dispatch


def _ffn_body(disp_ref, w1_ref, b1_ref, w2_ref, b2_ref, out_ref, d16_ref):
    f = pl.program_id(1)

    @pl.when(f == 0)
    def _():
        d16_ref[...] = disp_ref[...].astype(jnp.bfloat16)
        out_ref[...] = jnp.broadcast_to(b2_ref[...].reshape(1, 1, HID), (1, C, HID))

    h = jnp.dot(d16_ref[...], w1_ref[...][0].astype(jnp.bfloat16),
                preferred_element_type=jnp.float32)
    h = jnp.maximum(h + b1_ref[...][0], 0.0)
    acc = jnp.dot(h.astype(jnp.bfloat16), w2_ref[...][0].astype(jnp.bfloat16),
                  preferred_element_type=jnp.float32)
    out_ref[...] += acc[None]


def _ffn(disp, W1, b1, W2, b2):
    return pl.pallas_call(
        _ffn_body,
        grid=(E, NF),
        in_specs=[
            pl.BlockSpec((C, HID), lambda e, f: (e, 0)),
            pl.BlockSpec((1, HID, FBLK), lambda e, f: (e, 0, f)),
            pl.BlockSpec((1, 1, FBLK), lambda e, f: (e, 0, f)),
            pl.BlockSpec((1, FBLK, HID), lambda e, f: (e, f, 0)),
            pl.BlockSpec((1, 1, HID), lambda e, f: (e, 0, 0)),
        ],
        out_specs=pl.BlockSpec((1, C, HID), lambda e, f: (e, 0, 0)),
        out_shape=jax.ShapeDtypeStruct((E, C, HID), jnp.float32),
        scratch_shapes=[pltpu.VMEM((C, HID), jnp.bfloat16)],
    )(disp, W1, b1.reshape(E, 1, FF), W2, b2.reshape(E, 1, HID))


@functools.cache
def _combine_sc():
    @functools.partial(
        pl.kernel,
        mesh=_sc_mesh(),
        out_type=jax.ShapeDtypeStruct((T, HID), jnp.float32),
        scratch_types=[
            pltpu.VMEM((_DCHUNK,), jnp.int32),
            pltpu.VMEM((_DCHUNK, 16), jnp.float32),
            pltpu.VMEM((_DCHUNK, HID), jnp.float32),
            pltpu.SemaphoreType.DMA,
        ],
    )
    def combine(eo_hbm, src_hbm, gate_hbm, out_hbm, idx_v, gate_v, rows_v, sem):
        wid = lax.axis_index("s") * 2 + lax.axis_index("c")
        for k in range(TPT // _DCHUNK):
            base = wid * TPT + k * _DCHUNK
            pltpu.sync_copy(src_hbm.at[pl.ds(base, _DCHUNK)], idx_v)
            pltpu.sync_copy(gate_hbm.at[pl.ds(base, _DCHUNK)], gate_v)
            pltpu.async_copy(eo_hbm.at[idx_v], rows_v, sem).wait()

            def scale(j, _):
                g = gate_v[j, :]
                for c0 in range(0, HID, 16):
                    rows_v[j, pl.ds(c0, 16)] = rows_v[j, pl.ds(c0, 16)] * g
                return _

            lax.fori_loop(0, _DCHUNK, scale, 0)
            pltpu.sync_copy(rows_v, out_hbm.at[pl.ds(base, _DCHUNK)])

    return combine


def kernel(x, Wg, W1, b1, W2, b2):
    B, S, D = x.shape
    xt = x.reshape(T, D)
    dest, src, gate_rep = _routing(xt, Wg)
    disp = _dispatch_sc()(xt, dest)
    eo = _ffn(disp, W1, b1, W2, b2)
    out = _combine_sc()(eo.reshape(T, D), src, gate_rep)
    return out.reshape(B, S, D)


# speedup vs baseline: 1.1360x; 1.1034x over previous
import functools

import jax
import jax.numpy as jnp
from jax import lax
from jax.experimental import pallas as pl
from jax.experimental.pallas import tpu as pltpu
from jax.experimental.pallas import tpu_sc as plsc

HID = 1024
FF = 4096
E = 8
T = 4096
C = T // E
CHUNK = 512
NCHUNK = T // CHUNK
NTILES = 32
TPT = T // NTILES
DISP_ROWS = T + NTILES
FBLK = 1024
NF = FF // FBLK


def _routing_body(x_ref, wg_ref, dest_ref, src_ref, gate_ref, cnt_ref):
    pid = pl.program_id(0)

    @pl.when(pid == 0)
    def _():
        cnt_ref[...] = jnp.zeros_like(cnt_ref)

    xb = x_ref[...].astype(jnp.bfloat16)
    wg = wg_ref[...].astype(jnp.bfloat16)
    logits = jnp.dot(xb, wg, preferred_element_type=jnp.float32)
    m = jnp.max(logits, axis=1, keepdims=True)
    gate = 1.0 / jnp.sum(jnp.exp(logits - m), axis=1, keepdims=True)
    iota_e = lax.broadcasted_iota(jnp.int32, (CHUNK, E), 1)
    eidx = jnp.min(jnp.where(logits == m, iota_e, E), axis=1, keepdims=True)
    maskf = (iota_e == eidx).astype(jnp.float32)

    r = lax.broadcasted_iota(jnp.int32, (CHUNK, CHUNK), 0)
    c = lax.broadcasted_iota(jnp.int32, (CHUNK, CHUNK), 1)
    l16 = (c < r).astype(jnp.bfloat16)
    locin = jnp.dot(l16, maskf.astype(jnp.bfloat16),
                    preferred_element_type=jnp.float32)
    base = cnt_ref[0:1, 0:E]
    locf = jnp.sum((locin + base) * maskf, axis=1, keepdims=True)
    cnt_ref[0:1, 0:E] = base + jnp.sum(maskf, axis=0, keepdims=True)

    keep = locf < float(C)
    loc = locf.astype(jnp.int32)
    locc = jnp.minimum(loc, C - 1)
    tok = pid * CHUNK + lax.broadcasted_iota(jnp.int32, (CHUNK, 1), 0)
    trash = T + tok // TPT
    dest_ref[...] = jnp.where(keep, eidx * C + loc, trash)
    src_ref[...] = eidx * C + locc
    gate_ref[...] = jnp.broadcast_to(gate * keep.astype(jnp.float32), (CHUNK, 16))


def _routing(xt, Wg):
    out_shape = (
        jax.ShapeDtypeStruct((T, 1), jnp.int32),
        jax.ShapeDtypeStruct((T, 1), jnp.int32),
        jax.ShapeDtypeStruct((T, 16), jnp.float32),
    )
    dest, src, gate_rep = pl.pallas_call(
        _routing_body,
        grid=(NCHUNK,),
        in_specs=[
            pl.BlockSpec((CHUNK, HID), lambda i: (i, 0)),
            pl.BlockSpec((HID, E), lambda i: (0, 0)),
        ],
        out_specs=(
            pl.BlockSpec((CHUNK, 1), lambda i: (i, 0)),
            pl.BlockSpec((CHUNK, 1), lambda i: (i, 0)),
            pl.BlockSpec((CHUNK, 16), lambda i: (i, 0)),
        ),
        out_shape=out_shape,
        scratch_shapes=[pltpu.VMEM((8, 128), jnp.float32)],
    )(xt, Wg)
    return dest.reshape(T), src.reshape(T), gate_rep


_DCHUNK = 64


@functools.cache
def _sc_mesh():
    return plsc.VectorSubcoreMesh(core_axis_name="c", subcore_axis_name="s")


@functools.cache
def _dispatch_sc():
    @functools.partial(
        pl.kernel,
        mesh=_sc_mesh(),
        out_type=jax.ShapeDtypeStruct((DISP_ROWS, HID), jnp.float32),
        scratch_types=[
            pltpu.VMEM((_DCHUNK,), jnp.int32),
            pltpu.VMEM((_DCHUNK, HID), jnp.float32),
            pltpu.SemaphoreType.DMA,
        ],
    )
    def dispatch(xt_hbm, dest_hbm, disp_hbm, idx_v, rows_v, sem):
        wid = lax.axis_index("s") * 2 + lax.axis_index("c")
        for k in range(TPT // _DCHUNK):
            base = wid * TPT + k * _DCHUNK
            pltpu.sync_copy(xt_hbm.at[pl.ds(base, _DCHUNK)], rows_v)
            pltpu.sync_copy(dest_hbm.at[pl.ds(base, _DCHUNK)], idx_v)
            pltpu.async_copy(rows_v, disp_hbm.at[idx_v], sem).wait()

    return dispatch


def _ffn_body(disp_ref, w1_ref, b1_ref, w2_ref, b2_ref, out_ref, d16_ref):
    f = pl.program_id(1)

    @pl.when(f == 0)
    def _():
        d16_ref[...] = disp_ref[...].astype(jnp.bfloat16)
        out_ref[...] = jnp.broadcast_to(b2_ref[...].reshape(1, 1, HID), (1, C, HID))

    h = jnp.dot(d16_ref[...], w1_ref[...][0].astype(jnp.bfloat16),
                preferred_element_type=jnp.float32)
    h = jnp.maximum(h + b1_ref[...][0], 0.0)
    acc = jnp.dot(h.astype(jnp.bfloat16), w2_ref[...][0].astype(jnp.bfloat16),
                  preferred_element_type=jnp.float32)
    out_ref[...] += acc[None]


def _ffn(disp, W1, b1, W2, b2):
    return pl.pallas_call(
        _ffn_body,
        grid=(E, NF),
        in_specs=[
            pl.BlockSpec((C, HID), lambda e, f: (e, 0)),
            pl.BlockSpec((1, HID, FBLK), lambda e, f: (e, 0, f)),
            pl.BlockSpec((1, 1, FBLK), lambda e, f: (e, 0, f)),
            pl.BlockSpec((1, FBLK, HID), lambda e, f: (e, f, 0)),
            pl.BlockSpec((1, 1, HID), lambda e, f: (e, 0, 0)),
        ],
        out_specs=pl.BlockSpec((1, C, HID), lambda e, f: (e, 0, 0)),
        out_shape=jax.ShapeDtypeStruct((E, C, HID), jnp.float32),
        scratch_shapes=[pltpu.VMEM((C, HID), jnp.bfloat16)],
    )(disp, W1, b1.reshape(E, 1, FF), W2, b2.reshape(E, 1, HID))


@functools.cache
def _combine_sc():
    @functools.partial(
        pl.kernel,
        mesh=_sc_mesh(),
        out_type=jax.ShapeDtypeStruct((T, HID), jnp.float32),
        scratch_types=[
            pltpu.VMEM((_DCHUNK,), jnp.int32),
            pltpu.VMEM((_DCHUNK, 16), jnp.float32),
            pltpu.VMEM((_DCHUNK, HID), jnp.float32),
            pltpu.SemaphoreType.DMA,
        ],
    )
    def combine(eo_hbm, src_hbm, gate_hbm, out_hbm, idx_v, gate_v, rows_v, sem):
        wid = lax.axis_index("s") * 2 + lax.axis_index("c")
        for k in range(TPT // _DCHUNK):
            base = wid * TPT + k * _DCHUNK
            pltpu.sync_copy(src_hbm.at[pl.ds(base, _DCHUNK)], idx_v)
            pltpu.sync_copy(gate_hbm.at[pl.ds(base, _DCHUNK)], gate_v)
            pltpu.async_copy(eo_hbm.at[idx_v], rows_v, sem).wait()

            def scale(j, _):
                g = gate_v[j, :]
                for c0 in range(0, HID, 16):
                    rows_v[j, pl.ds(c0, 16)] = rows_v[j, pl.ds(c0, 16)] * g
                return _

            lax.fori_loop(0, _DCHUNK, scale, 0)
            pltpu.sync_copy(rows_v, out_hbm.at[pl.ds(base, _DCHUNK)])

    return combine


def kernel(x, Wg, W1, b1, W2, b2):
    B, S, D = x.shape
    xt = x.reshape(T, D)
    dest, src, gate_rep = _routing(xt, Wg)
    disp = _dispatch_sc()(xt, dest)
    eo = _ffn(disp, W1, b1, W2, b2)
    out = _combine_sc()(eo.reshape(T, D), src, gate_rep)
    return out.reshape(B, S, D)


# speedup vs baseline: 1.1635x; 1.0242x over previous
import functools

import jax
import jax.numpy as jnp
from jax import lax
from jax.experimental import pallas as pl
from jax.experimental.pallas import tpu as pltpu
from jax.experimental.pallas import tpu_sc as plsc

HID = 1024
FF = 4096
E = 8
T = 4096
C = T // E
CHUNK = 512
NCHUNK = T // CHUNK
NTILES = 32
TPT = T // NTILES
DISP_ROWS = T + NTILES
FBLK = 2048
NF = FF // FBLK


def _routing_body(x_ref, wg_ref, dest_ref, src_ref, gate_ref, cnt_ref):
    pid = pl.program_id(0)

    @pl.when(pid == 0)
    def _():
        cnt_ref[...] = jnp.zeros_like(cnt_ref)

    xb = x_ref[...].astype(jnp.bfloat16)
    wg = wg_ref[...].astype(jnp.bfloat16)
    logits = jnp.dot(xb, wg, preferred_element_type=jnp.float32)
    m = jnp.max(logits, axis=1, keepdims=True)
    gate = 1.0 / jnp.sum(jnp.exp(logits - m), axis=1, keepdims=True)
    iota_e = lax.broadcasted_iota(jnp.int32, (CHUNK, E), 1)
    eidx = jnp.min(jnp.where(logits == m, iota_e, E), axis=1, keepdims=True)
    maskf = (iota_e == eidx).astype(jnp.float32)

    r = lax.broadcasted_iota(jnp.int32, (CHUNK, CHUNK), 0)
    c = lax.broadcasted_iota(jnp.int32, (CHUNK, CHUNK), 1)
    l16 = (c < r).astype(jnp.bfloat16)
    locin = jnp.dot(l16, maskf.astype(jnp.bfloat16),
                    preferred_element_type=jnp.float32)
    base = cnt_ref[0:1, 0:E]
    locf = jnp.sum((locin + base) * maskf, axis=1, keepdims=True)
    cnt_ref[0:1, 0:E] = base + jnp.sum(maskf, axis=0, keepdims=True)

    keep = locf < float(C)
    loc = locf.astype(jnp.int32)
    locc = jnp.minimum(loc, C - 1)
    tok = pid * CHUNK + lax.broadcasted_iota(jnp.int32, (CHUNK, 1), 0)
    trash = T + tok // TPT
    dest_ref[...] = jnp.where(keep, eidx * C + loc, trash)
    src_ref[...] = eidx * C + locc
    gate_ref[...] = jnp.broadcast_to(gate * keep.astype(jnp.float32), (CHUNK, 16))


def _routing(xt, Wg):
    out_shape = (
        jax.ShapeDtypeStruct((T, 1), jnp.int32),
        jax.ShapeDtypeStruct((T, 1), jnp.int32),
        jax.ShapeDtypeStruct((T, 16), jnp.float32),
    )
    dest, src, gate_rep = pl.pallas_call(
        _routing_body,
        grid=(NCHUNK,),
        in_specs=[
            pl.BlockSpec((CHUNK, HID), lambda i: (i, 0)),
            pl.BlockSpec((HID, E), lambda i: (0, 0)),
        ],
        out_specs=(
            pl.BlockSpec((CHUNK, 1), lambda i: (i, 0)),
            pl.BlockSpec((CHUNK, 1), lambda i: (i, 0)),
            pl.BlockSpec((CHUNK, 16), lambda i: (i, 0)),
        ),
        out_shape=out_shape,
        scratch_shapes=[pltpu.VMEM((8, 128), jnp.float32)],
    )(xt, Wg)
    return dest.reshape(T), src.reshape(T), gate_rep


_DCHUNK = 64


@functools.cache
def _sc_mesh():
    return plsc.VectorSubcoreMesh(core_axis_name="c", subcore_axis_name="s")


@functools.cache
def _dispatch_sc():
    @functools.partial(
        pl.kernel,
        mesh=_sc_mesh(),
        out_type=jax.ShapeDtypeStruct((DISP_ROWS, HID), jnp.float32),
        scratch_types=[
            pltpu.VMEM((_DCHUNK,), jnp.int32),
            pltpu.VMEM((_DCHUNK, HID), jnp.float32),
            pltpu.SemaphoreType.DMA,
        ],
    )
    def dispatch(xt_hbm, dest_hbm, disp_hbm, idx_v, rows_v, sem):
        wid = lax.axis_index("s") * 2 + lax.axis_index("c")
        for k in range(TPT // _DCHUNK):
            base = wid * TPT + k * _DCHUNK
            pltpu.sync_copy(xt_hbm.at[pl.ds(base, _DCHUNK)], rows_v)
            pltpu.sync_copy(dest_hbm.at[pl.ds(base, _DCHUNK)], idx_v)
            pltpu.async_copy(rows_v, disp_hbm.at[idx_v], sem).wait()

    return dispatch


def _ffn_body(disp_ref, w1_ref, b1_ref, w2_ref, b2_ref, out_ref, d16_ref):
    f = pl.program_id(1)

    @pl.when(f == 0)
    def _():
        d16_ref[...] = disp_ref[...].astype(jnp.bfloat16)
        out_ref[...] = jnp.broadcast_to(b2_ref[...].reshape(1, 1, HID), (1, C, HID))

    h = jnp.dot(d16_ref[...], w1_ref[...][0].astype(jnp.bfloat16),
                preferred_element_type=jnp.float32)
    h = jnp.maximum(h + b1_ref[...][0], 0.0)
    acc = jnp.dot(h.astype(jnp.bfloat16), w2_ref[...][0].astype(jnp.bfloat16),
                  preferred_element_type=jnp.float32)
    out_ref[...] += acc[None]


def _ffn(disp, W1, b1, W2, b2):
    return pl.pallas_call(
        _ffn_body,
        grid=(E, NF),
        in_specs=[
            pl.BlockSpec((C, HID), lambda e, f: (e, 0)),
            pl.BlockSpec((1, HID, FBLK), lambda e, f: (e, 0, f)),
            pl.BlockSpec((1, 1, FBLK), lambda e, f: (e, 0, f)),
            pl.BlockSpec((1, FBLK, HID), lambda e, f: (e, f, 0)),
            pl.BlockSpec((1, 1, HID), lambda e, f: (e, 0, 0)),
        ],
        out_specs=pl.BlockSpec((1, C, HID), lambda e, f: (e, 0, 0)),
        out_shape=jax.ShapeDtypeStruct((E, C, HID), jnp.float32),
        scratch_shapes=[pltpu.VMEM((C, HID), jnp.bfloat16)],
    )(disp, W1, b1.reshape(E, 1, FF), W2, b2.reshape(E, 1, HID))


@functools.cache
def _combine_sc():
    @functools.partial(
        pl.kernel,
        mesh=_sc_mesh(),
        out_type=jax.ShapeDtypeStruct((T, HID), jnp.float32),
        scratch_types=[
            pltpu.VMEM((_DCHUNK,), jnp.int32),
            pltpu.VMEM((_DCHUNK, 16), jnp.float32),
            pltpu.VMEM((_DCHUNK, HID), jnp.float32),
            pltpu.SemaphoreType.DMA,
        ],
    )
    def combine(eo_hbm, src_hbm, gate_hbm, out_hbm, idx_v, gate_v, rows_v, sem):
        wid = lax.axis_index("s") * 2 + lax.axis_index("c")
        for k in range(TPT // _DCHUNK):
            base = wid * TPT + k * _DCHUNK
            pltpu.sync_copy(src_hbm.at[pl.ds(base, _DCHUNK)], idx_v)
            pltpu.sync_copy(gate_hbm.at[pl.ds(base, _DCHUNK)], gate_v)
            pltpu.async_copy(eo_hbm.at[idx_v], rows_v, sem).wait()

            def scale(j, _):
                g = gate_v[j, :]
                for c0 in range(0, HID, 16):
                    rows_v[j, pl.ds(c0, 16)] = rows_v[j, pl.ds(c0, 16)] * g
                return _

            lax.fori_loop(0, _DCHUNK, scale, 0)
            pltpu.sync_copy(rows_v, out_hbm.at[pl.ds(base, _DCHUNK)])

    return combine


def kernel(x, Wg, W1, b1, W2, b2):
    B, S, D = x.shape
    xt = x.reshape(T, D)
    dest, src, gate_rep = _routing(xt, Wg)
    disp = _dispatch_sc()(xt, dest)
    eo = _ffn(disp, W1, b1, W2, b2)
    out = _combine_sc()(eo.reshape(T, D), src, gate_rep)
    return out.reshape(B, S, D)


# speedup vs baseline: 1.1953x; 1.0274x over previous
import functools

import jax
import jax.numpy as jnp
from jax import lax
from jax.experimental import pallas as pl
from jax.experimental.pallas import tpu as pltpu
from jax.experimental.pallas import tpu_sc as plsc

HID = 1024
FF = 4096
E = 8
T = 4096
C = T // E
CHUNK = 512
NCHUNK = T // CHUNK
NTILES = 32
TPT = T // NTILES
DISP_ROWS = T + NTILES
FBLK = 2048
NF = FF // FBLK


def _routing_body(x_ref, wg_ref, dest_ref, src_ref, gate_ref, cnt_ref):
    pid = pl.program_id(0)

    @pl.when(pid == 0)
    def _():
        cnt_ref[...] = jnp.zeros_like(cnt_ref)

    xb = x_ref[...].astype(jnp.bfloat16)
    wg = wg_ref[...].astype(jnp.bfloat16)
    logits = jnp.dot(xb, wg, preferred_element_type=jnp.float32)
    m = jnp.max(logits, axis=1, keepdims=True)
    gate = 1.0 / jnp.sum(jnp.exp(logits - m), axis=1, keepdims=True)
    iota_e = lax.broadcasted_iota(jnp.int32, (CHUNK, E), 1)
    eidx = jnp.min(jnp.where(logits == m, iota_e, E), axis=1, keepdims=True)
    maskf = (iota_e == eidx).astype(jnp.float32)

    r = lax.broadcasted_iota(jnp.int32, (CHUNK, CHUNK), 0)
    c = lax.broadcasted_iota(jnp.int32, (CHUNK, CHUNK), 1)
    l16 = (c < r).astype(jnp.bfloat16)
    locin = jnp.dot(l16, maskf.astype(jnp.bfloat16),
                    preferred_element_type=jnp.float32)
    base = cnt_ref[0:1, 0:E]
    locf = jnp.sum((locin + base) * maskf, axis=1, keepdims=True)
    cnt_ref[0:1, 0:E] = base + jnp.sum(maskf, axis=0, keepdims=True)

    keep = locf < float(C)
    loc = locf.astype(jnp.int32)
    locc = jnp.minimum(loc, C - 1)
    tok = pid * CHUNK + lax.broadcasted_iota(jnp.int32, (CHUNK, 1), 0)
    trash = T + tok // TPT
    dest_ref[...] = jnp.where(keep, eidx * C + loc, trash)
    src_ref[...] = eidx * C + locc
    gate_ref[...] = jnp.broadcast_to(gate * keep.astype(jnp.float32), (CHUNK, 16))


def _routing(xt, Wg):
    out_shape = (
        jax.ShapeDtypeStruct((T, 1), jnp.int32),
        jax.ShapeDtypeStruct((T, 1), jnp.int32),
        jax.ShapeDtypeStruct((T, 16), jnp.float32),
    )
    dest, src, gate_rep = pl.pallas_call(
        _routing_body,
        grid=(NCHUNK,),
        in_specs=[
            pl.BlockSpec((CHUNK, HID), lambda i: (i, 0)),
            pl.BlockSpec((HID, E), lambda i: (0, 0)),
        ],
        out_specs=(
            pl.BlockSpec((CHUNK, 1), lambda i: (i, 0)),
            pl.BlockSpec((CHUNK, 1), lambda i: (i, 0)),
            pl.BlockSpec((CHUNK, 16), lambda i: (i, 0)),
        ),
        out_shape=out_shape,
        scratch_shapes=[pltpu.VMEM((8, 128), jnp.float32)],
    )(xt, Wg)
    return dest.reshape(T), src.reshape(T), gate_rep


_DCHUNK = 64


@functools.cache
def _sc_mesh():
    return plsc.VectorSubcoreMesh(core_axis_name="c", subcore_axis_name="s")


@functools.cache
def _dispatch_sc():
    @functools.partial(
        pl.kernel,
        mesh=_sc_mesh(),
        out_type=jax.ShapeDtypeStruct((DISP_ROWS, HID), jnp.float32),
        scratch_types=[
            pltpu.VMEM((_DCHUNK,), jnp.int32),
            pltpu.VMEM((_DCHUNK, HID), jnp.float32),
            pltpu.SemaphoreType.DMA,
        ],
    )
    def dispatch(xt_hbm, dest_hbm, disp_hbm, idx_v, rows_v, sem):
        wid = lax.axis_index("s") * 2 + lax.axis_index("c")
        for k in range(TPT // _DCHUNK):
            base = wid * TPT + k * _DCHUNK
            pltpu.sync_copy(xt_hbm.at[pl.ds(base, _DCHUNK)], rows_v)
            pltpu.sync_copy(dest_hbm.at[pl.ds(base, _DCHUNK)], idx_v)
            pltpu.async_copy(rows_v, disp_hbm.at[idx_v], sem).wait()

    return dispatch


def _ffn_body(disp_ref, w1_ref, b1_ref, w2_ref, b2_ref, out_ref, d16_ref):
    f = pl.program_id(1)

    @pl.when(f == 0)
    def _():
        d16_ref[...] = disp_ref[...].astype(jnp.bfloat16)
        out_ref[...] = jnp.broadcast_to(b2_ref[...].reshape(1, 1, HID), (1, C, HID))

    h = jnp.dot(d16_ref[...], w1_ref[...][0].astype(jnp.bfloat16),
                preferred_element_type=jnp.float32)
    h = jnp.maximum(h + b1_ref[...][0], 0.0)
    acc = jnp.dot(h.astype(jnp.bfloat16), w2_ref[...][0].astype(jnp.bfloat16),
                  preferred_element_type=jnp.float32)
    out_ref[...] += acc[None]


def _ffn(disp, W1, b1, W2, b2):
    return pl.pallas_call(
        _ffn_body,
        grid=(E, NF),
        in_specs=[
            pl.BlockSpec((C, HID), lambda e, f: (e, 0)),
            pl.BlockSpec((1, HID, FBLK), lambda e, f: (e, 0, f)),
            pl.BlockSpec((1, 1, FBLK), lambda e, f: (e, 0, f)),
            pl.BlockSpec((1, FBLK, HID), lambda e, f: (e, f, 0)),
            pl.BlockSpec((1, 1, HID), lambda e, f: (e, 0, 0)),
        ],
        out_specs=pl.BlockSpec((1, C, HID), lambda e, f: (e, 0, 0)),
        out_shape=jax.ShapeDtypeStruct((E, C, HID), jnp.float32),
        scratch_shapes=[pltpu.VMEM((C, HID), jnp.bfloat16)],
    )(disp, W1, b1.reshape(E, 1, FF), W2, b2.reshape(E, 1, HID))


_CCHUNK = 32


@functools.cache
def _combine_sc():
    @functools.partial(
        pl.kernel,
        mesh=_sc_mesh(),
        out_type=jax.ShapeDtypeStruct((T, HID), jnp.float32),
        scratch_types=[
            pltpu.VMEM((TPT,), jnp.int32),
            pltpu.VMEM((TPT, 16), jnp.float32),
            pltpu.VMEM((_CCHUNK, HID), jnp.float32),
            pltpu.VMEM((_CCHUNK, HID), jnp.float32),
            pltpu.SemaphoreType.DMA,
            pltpu.SemaphoreType.DMA,
            pltpu.SemaphoreType.DMA,
            pltpu.SemaphoreType.DMA,
        ],
    )
    def combine(eo_hbm, src_hbm, gate_hbm, out_hbm, idx_v, gate_v,
                rows0, rows1, g0, g1, w0, w1):
        wid = lax.axis_index("s") * 2 + lax.axis_index("c")
        base = wid * TPT
        pltpu.sync_copy(src_hbm.at[pl.ds(base, TPT)], idx_v)
        pltpu.sync_copy(gate_hbm.at[pl.ds(base, TPT)], gate_v)
        NR = TPT // _CCHUNK
        bufs, gsems, wsems = (rows0, rows1), (g0, g1), (w0, w1)
        gathers = [None] * NR
        writes = [None] * NR
        gathers[0] = pltpu.async_copy(
            eo_hbm.at[idx_v.at[pl.ds(0, _CCHUNK)]], bufs[0], gsems[0])
        for k in range(NR):
            b = k % 2
            if k + 1 < NR:
                if k >= 1:
                    writes[k - 1].wait()
                gathers[k + 1] = pltpu.async_copy(
                    eo_hbm.at[idx_v.at[pl.ds((k + 1) * _CCHUNK, _CCHUNK)]],
                    bufs[1 - b], gsems[1 - b])
            gathers[k].wait()

            def scale(j, _, _k=k, _b=b):
                g = gate_v[_k * _CCHUNK + j, :]
                for c0 in range(0, HID, 16):
                    bufs[_b][j, pl.ds(c0, 16)] = bufs[_b][j, pl.ds(c0, 16)] * g
                return _

            lax.fori_loop(0, _CCHUNK, scale, 0)
            writes[k] = pltpu.async_copy(
                bufs[b], out_hbm.at[pl.ds(base + k * _CCHUNK, _CCHUNK)], wsems[b])
        writes[NR - 2].wait()
        writes[NR - 1].wait()

    return combine


def kernel(x, Wg, W1, b1, W2, b2):
    B, S, D = x.shape
    xt = x.reshape(T, D)
    dest, src, gate_rep = _routing(xt, Wg)
    disp = _dispatch_sc()(xt, dest)
    eo = _ffn(disp, W1, b1, W2, b2)
    out = _combine_sc()(eo.reshape(T, D), src, gate_rep)
    return out.reshape(B, S, D)
